# baseline (speedup 1.0000x reference)
import functools

import jax
import jax.numpy as jnp
from jax import lax
from jax.experimental import pallas as pl
from jax.experimental.pallas import tpu as pltpu
from jax.experimental.pallas import tpu_sc as plsc

HASH_SIZE = 128
OFF_SIZE = 64
FEATS = 16
N_QUERIES = 1048576

NUM_WORKERS = 32
PER_WORKER = N_QUERIES // NUM_WORKERS
CHUNK = 1024
GATHERS = CHUNK // 128
N_CHUNKS = PER_WORKER // CHUNK
QT = CHUNK // 128
FB_STRIDE = (N_QUERIES // 128) * 8 * 128

_mesh = plsc.VectorSubcoreMesh(core_axis_name="c", subcore_axis_name="s")


@functools.partial(
    pl.kernel,
    mesh=_mesh,
    compiler_params=pltpu.CompilerParams(use_tc_tiling_on_sc=False,
                                         needs_layout_passes=False),
    out_type=jax.ShapeDtypeStruct((2 * FB_STRIDE,), jnp.float32),
    scratch_types=[
        pltpu.VMEM((CHUNK,), jnp.int32),
        pltpu.VMEM((CHUNK,), jnp.int32),
        pltpu.VMEM((CHUNK,), jnp.int32),
        pltpu.VMEM((2, CHUNK), jnp.int32),
        pltpu.VMEM((2, CHUNK), jnp.int32),
        pltpu.VMEM((2, CHUNK, FEATS), jnp.float32),
        pltpu.VMEM((2 * QT * 8 * 128,), jnp.float32),
        pltpu.VMEM((2, GATHERS, 128), jnp.int32),
        pltpu.VMEM((2, GATHERS, 128), jnp.int32),
        pltpu.VMEM((3, 16), jnp.float32),
        pltpu.VMEM((3, 16), jnp.float32),
        pltpu.VMEM_SHARED((OFF_SIZE ** 3,), jnp.int32),
        pltpu.SemaphoreType.DMA,
        pltpu.SemaphoreType.DMA,
        pltpu.SemaphoreType.DMA,
        pltpu.SemaphoreType.DMA,
    ],
)
def _psh_sc(coords_t_hbm, hashf_hbm, offp_hbm, m0_hbm, m1_hbm, out_hbm,
            c0_v, c1_v, c2_v, hp_v, offw_v, feats_v, tbuf_v, oidx_v, hidx_v,
            m0_v, m1_v, offsp_v, semo0, semo1, semf0, semf1):
    wid = lax.axis_index("c") * 16 + lax.axis_index("s")
    base = wid * PER_WORKER
    sid = lax.axis_index("s")
    seg = OFF_SIZE ** 3 // 16
    pltpu.sync_copy(offp_hbm.at[pl.ds(sid * seg, seg)],
                    offsp_v.at[pl.ds(sid * seg, seg)])
    plsc.subcore_barrier()
    pltpu.sync_copy(m0_hbm, m0_v)
    pltpu.sync_copy(m1_hbm, m1_v)
    cvs = (c0_v, c1_v, c2_v)
    semo = (semo0, semo1)
    semf = (semf0, semf1)
    lanes = lax.iota(jnp.int32, 16)
    faddr = (lanes >> 3) * (QT * 8 * 128) + (lanes & 7) * 128

    def stage1(t, p):
        row0 = base + t * CHUNK

        for d in range(3):
            pltpu.sync_copy(coords_t_hbm.at[pl.ds(d * N_QUERIES + row0, CHUNK)],
                            cvs[d])

        def pass_a(g, carry_a):
            for k in range(8):
                q0 = g * 128 + k * 16
                oi, hpc = [], []
                for d in range(3):
                    cf = cvs[d][pl.ds(q0, 16)].astype(jnp.float32)
                    oi.append((cf * m1_v[d]).astype(jnp.int32) & (OFF_SIZE - 1))
                    hpc.append((cf * m0_v[d]).astype(jnp.int32) & (HASH_SIZE - 1))
                oidx_v[p, g, pl.ds(k * 16, 16)] = (oi[0] << 12) | (oi[1] << 6) | oi[2]
                hp_v[p, pl.ds(q0, 16)] = (hpc[0] << 20) | (hpc[1] << 10) | hpc[2]
            return carry_a

        lax.fori_loop(0, GATHERS, pass_a, 0)

    def fire_offsets(p):
        for g in range(GATHERS):
            pltpu.async_copy(offp_hbm.at[oidx_v.at[p, g]],
                             offw_v.at[p, pl.ds(g * 128, 128)], semo[p])

    def stage3(p):
        pltpu.make_async_copy(offp_hbm.at[pl.ds(0, CHUNK)], offw_v.at[p],
                              semo[p]).wait()

        def pass_b(g, carry_b):
            for k in range(8):
                q0 = g * 128 + k * 16
                s = hp_v[p, pl.ds(q0, 16)] + offw_v[p, pl.ds(q0, 16)]
                lin = (((s >> 20) & 127) << 14) | (((s >> 10) & 127) << 7) | (s & 127)
                hidx_v[p, g, pl.ds(k * 16, 16)] = lin
            return carry_b

        lax.fori_loop(0, GATHERS, pass_b, 0)

    def fire_feats(p):
        for g in range(GATHERS):
            pltpu.async_copy(hashf_hbm.at[hidx_v.at[p, g]],
                             feats_v.at[p, pl.ds(g * 128, 128)], semf[p])

    def stage5(t, p):
        pltpu.make_async_copy(hashf_hbm.at[pl.ds(0, CHUNK)], feats_v.at[p],
                              semf[p]).wait()

        def transpose_q(q, carry_t):
            row = feats_v[p, q, :]
            qt = q >> 7
            dst = faddr + (qt * 1024 + (q & 127))
            plsc.store_scatter(tbuf_v, [dst], row)
            return carry_t

        lax.fori_loop(0, CHUNK, transpose_q, 0)

        qt0 = (base + t * CHUNK) >> 7
        for fb in range(2):
            pltpu.sync_copy(
                tbuf_v.at[pl.ds(fb * QT * 8 * 128, QT * 8 * 128)],
                out_hbm.at[pl.ds(fb * FB_STRIDE + qt0 * 1024, QT * 8 * 128)])

    stage1(0, 0)
    fire_offsets(0)
    stage1(1, 1)
    fire_offsets(1)
    stage3(0)
    fire_feats(0)

    def body(i, carry):
        t = 2 * i
        stage1(t + 2, 0)
        fire_offsets(0)
        stage3(1)
        fire_feats(1)
        stage5(t, 0)
        stage1(t + 3, 1)
        fire_offsets(1)
        stage3(0)
        fire_feats(0)
        stage5(t + 1, 1)
        return carry

    lax.fori_loop(0, (N_CHUNKS - 2) // 2, body, 0)

    stage3(1)
    fire_feats(1)
    stage5(N_CHUNKS - 2, 0)
    stage5(N_CHUNKS - 1, 1)


def kernel(coords, hash_table, offset_table, m0, m1):
    hashf = hash_table.reshape(HASH_SIZE ** 3, FEATS)
    off3 = offset_table.reshape(OFF_SIZE ** 3, 3)
    offp = (off3[:, 0] << 20) | (off3[:, 1] << 10) | off3[:, 2]
    coords_t = coords.T.reshape(-1)
    m0b = jnp.broadcast_to(m0.reshape(3, 1), (3, 16))
    m1b = jnp.broadcast_to(m1.reshape(3, 1), (3, 16))
    out1d = _psh_sc(coords_t, hashf, offp, m0b, m1b)
    out4d = out1d.reshape(2, N_QUERIES // 128, 8, 128)
    return out4d.transpose(1, 3, 0, 2).reshape(N_QUERIES, FEATS)


# speedup vs baseline: 1.0215x; 1.0215x over previous
import functools

import jax
import jax.numpy as jnp
from jax import lax
from jax.experimental import pallas as pl
from jax.experimental.pallas import tpu as pltpu
from jax.experimental.pallas import tpu_sc as plsc

HASH_SIZE = 128
OFF_SIZE = 64
FEATS = 16
N_QUERIES = 1048576

NUM_WORKERS = 32
PER_WORKER = N_QUERIES // NUM_WORKERS
CHUNK = 2048
GATHERS = CHUNK // 128
N_CHUNKS = PER_WORKER // CHUNK
QT = CHUNK // 128
FB_STRIDE = (N_QUERIES // 128) * 8 * 128

_mesh = plsc.VectorSubcoreMesh(core_axis_name="c", subcore_axis_name="s")


@functools.partial(
    pl.kernel,
    mesh=_mesh,
    compiler_params=pltpu.CompilerParams(use_tc_tiling_on_sc=False,
                                         needs_layout_passes=False),
    out_type=jax.ShapeDtypeStruct((2 * FB_STRIDE,), jnp.float32),
    scratch_types=[
        pltpu.VMEM((CHUNK,), jnp.int32),
        pltpu.VMEM((CHUNK,), jnp.int32),
        pltpu.VMEM((CHUNK,), jnp.int32),
        pltpu.VMEM((2, CHUNK), jnp.int32),
        pltpu.VMEM((2, CHUNK), jnp.int32),
        pltpu.VMEM((2, CHUNK, FEATS), jnp.float32),
        pltpu.VMEM((2 * QT * 8 * 128,), jnp.float32),
        pltpu.VMEM((2, GATHERS, 128), jnp.int32),
        pltpu.VMEM((2, GATHERS, 128), jnp.int32),
        pltpu.VMEM((3, 16), jnp.float32),
        pltpu.VMEM((3, 16), jnp.float32),
        pltpu.SemaphoreType.DMA,
        pltpu.SemaphoreType.DMA,
        pltpu.SemaphoreType.DMA,
        pltpu.SemaphoreType.DMA,
    ],
)
def _psh_sc(coords_t_hbm, hashf_hbm, offp_hbm, m0_hbm, m1_hbm, out_hbm,
            c0_v, c1_v, c2_v, hp_v, offw_v, feats_v, tbuf_v, oidx_v, hidx_v,
            m0_v, m1_v, semo0, semo1, semf0, semf1):
    wid = lax.axis_index("c") * 16 + lax.axis_index("s")
    base = wid * PER_WORKER
    pltpu.sync_copy(m0_hbm, m0_v)
    pltpu.sync_copy(m1_hbm, m1_v)
    cvs = (c0_v, c1_v, c2_v)
    semo = (semo0, semo1)
    semf = (semf0, semf1)
    lanes = lax.iota(jnp.int32, 16)
    faddr = (lanes >> 3) * (QT * 8 * 128) + (lanes & 7) * 128

    def stage1(t, p):
        row0 = base + t * CHUNK

        for d in range(3):
            pltpu.sync_copy(coords_t_hbm.at[pl.ds(d * N_QUERIES + row0, CHUNK)],
                            cvs[d])

        def pass_a(g, carry_a):
            for k in range(8):
                q0 = g * 128 + k * 16
                oi, hpc = [], []
                for d in range(3):
                    cf = cvs[d][pl.ds(q0, 16)].astype(jnp.float32)
                    oi.append((cf * m1_v[d]).astype(jnp.int32) & (OFF_SIZE - 1))
                    hpc.append((cf * m0_v[d]).astype(jnp.int32) & (HASH_SIZE - 1))
                oidx_v[p, g, pl.ds(k * 16, 16)] = (oi[0] << 12) | (oi[1] << 6) | oi[2]
                hp_v[p, pl.ds(q0, 16)] = (hpc[0] << 20) | (hpc[1] << 10) | hpc[2]
            return carry_a

        lax.fori_loop(0, GATHERS, pass_a, 0)

    def fire_offsets(p):
        for g in range(GATHERS):
            pltpu.async_copy(offp_hbm.at[oidx_v.at[p, g]],
                             offw_v.at[p, pl.ds(g * 128, 128)], semo[p])

    def stage3(p):
        pltpu.make_async_copy(offp_hbm.at[pl.ds(0, CHUNK)], offw_v.at[p],
                              semo[p]).wait()

        def pass_b(g, carry_b):
            for k in range(8):
                q0 = g * 128 + k * 16
                s = hp_v[p, pl.ds(q0, 16)] + offw_v[p, pl.ds(q0, 16)]
                lin = (((s >> 20) & 127) << 14) | (((s >> 10) & 127) << 7) | (s & 127)
                hidx_v[p, g, pl.ds(k * 16, 16)] = lin
            return carry_b

        lax.fori_loop(0, GATHERS, pass_b, 0)

    def fire_feats(p):
        for g in range(GATHERS):
            pltpu.async_copy(hashf_hbm.at[hidx_v.at[p, g]],
                             feats_v.at[p, pl.ds(g * 128, 128)], semf[p])

    def stage5(t, p):
        pltpu.make_async_copy(hashf_hbm.at[pl.ds(0, CHUNK)], feats_v.at[p],
                              semf[p]).wait()

        def transpose_q(qq, carry_t):
            q0 = qq * 8
            dst0 = faddr + ((q0 >> 7) * 1024 + (q0 & 127))
            for u in range(8):
                plsc.store_scatter(tbuf_v, [dst0 + u], feats_v[p, q0 + u, :])
            return carry_t

        lax.fori_loop(0, CHUNK // 8, transpose_q, 0)

        qt0 = (base + t * CHUNK) >> 7
        for fb in range(2):
            pltpu.sync_copy(
                tbuf_v.at[pl.ds(fb * QT * 8 * 128, QT * 8 * 128)],
                out_hbm.at[pl.ds(fb * FB_STRIDE + qt0 * 1024, QT * 8 * 128)])

    stage1(0, 0)
    fire_offsets(0)
    stage1(1, 1)
    fire_offsets(1)
    stage3(0)
    fire_feats(0)

    def body(i, carry):
        t = 2 * i
        stage1(t + 2, 0)
        fire_offsets(0)
        stage3(1)
        fire_feats(1)
        stage5(t, 0)
        stage1(t + 3, 1)
        fire_offsets(1)
        stage3(0)
        fire_feats(0)
        stage5(t + 1, 1)
        return carry

    lax.fori_loop(0, (N_CHUNKS - 2) // 2, body, 0)

    stage3(1)
    fire_feats(1)
    stage5(N_CHUNKS - 2, 0)
    stage5(N_CHUNKS - 1, 1)


def kernel(coords, hash_table, offset_table, m0, m1):
    hashf = hash_table.reshape(HASH_SIZE ** 3, FEATS)
    off3 = offset_table.reshape(OFF_SIZE ** 3, 3)
    offp = (off3[:, 0] << 20) | (off3[:, 1] << 10) | off3[:, 2]
    coords_t = coords.T.reshape(-1)
    m0b = jnp.broadcast_to(m0.reshape(3, 1), (3, 16))
    m1b = jnp.broadcast_to(m1.reshape(3, 1), (3, 16))
    out1d = _psh_sc(coords_t, hashf, offp, m0b, m1b)
    out4d = out1d.reshape(2, N_QUERIES // 128, 8, 128)
    return out4d.transpose(1, 3, 0, 2).reshape(N_QUERIES, FEATS)
